# baseline (speedup 1.0000x reference)
import math

import jax
import jax.numpy as jnp
from jax.experimental import pallas as pl
from jax.experimental.pallas import tpu as pltpu

B = 2
S = 2048
D = 1024
H = 16
DH = 64
POOL = 512
TOPK = 128

BLK_S = 512
BLK_Q = 512
BLK_K = 512


def _qkv_proj_kernel(x_ref, aqk_ref, av_ref, rq_ref, rk_ref, rv_ref,
                     q_ref, k_ref, v_ref):
    x = x_ref[0]
    h_qk = jax.lax.dot_general(x, aqk_ref[0], (((1,), (1,)), ((), ())),
                               preferred_element_type=jnp.float32)
    h_v = jax.lax.dot_general(x, av_ref[0], (((1,), (1,)), ((), ())),
                              preferred_element_type=jnp.float32)
    q_ref[0] = jnp.dot(h_qk, rq_ref[0],
                       preferred_element_type=jnp.float32).astype(jnp.bfloat16)
    k_ref[0] = jnp.dot(h_qk, rk_ref[0],
                       preferred_element_type=jnp.float32).astype(jnp.bfloat16)
    v_ref[0] = jnp.dot(h_v, rv_ref[0],
                       preferred_element_type=jnp.float32).astype(jnp.bfloat16)


def _flash_kernel(q_ref, k_ref, v_ref, o_ref, s_scr, p_scr):
    i = pl.program_id(1)

    @pl.when(i == 0)
    def _zero():
        p_scr[...] = jnp.zeros(p_scr.shape, jnp.bfloat16)

    tri = (jax.lax.broadcasted_iota(jnp.int32, (BLK_Q, BLK_K), 1) <=
           jax.lax.broadcasted_iota(jnp.int32, (BLK_Q, BLK_K), 0))

    for h in range(H):
        hs = slice(h * DH, (h + 1) * DH)
        qh = q_ref[0, :, hs]

        def p1(j, m, qh=qh, hs=hs):
            s = jax.lax.dot_general(qh, k_ref[0, pl.ds(j * BLK_K, BLK_K), hs],
                                    (((1,), (1,)), ((), ())),
                                    preferred_element_type=jnp.float32)
            s_scr[:, pl.ds(j * BLK_K, BLK_K)] = s
            return jnp.maximum(m, jnp.max(s, axis=1, keepdims=True))

        m = jax.lax.fori_loop(0, i, p1,
                              jnp.full((BLK_Q, 1), -jnp.inf, jnp.float32))
        s = jax.lax.dot_general(qh, k_ref[0, pl.ds(i * BLK_K, BLK_K), hs],
                                (((1,), (1,)), ((), ())),
                                preferred_element_type=jnp.float32)
        s = jnp.where(tri, s, -1e30)
        s_scr[:, pl.ds(i * BLK_K, BLK_K)] = s
        m = jnp.maximum(m, jnp.max(s, axis=1, keepdims=True))

        def p2(j, l, m=m):
            p = jnp.exp(s_scr[:, pl.ds(j * BLK_K, BLK_K)] - m)
            p_scr[:, pl.ds(j * BLK_K, BLK_K)] = p.astype(jnp.bfloat16)
            return l + jnp.sum(p, axis=1, keepdims=True)

        l = jax.lax.fori_loop(0, i + 1, p2,
                              jnp.zeros((BLK_Q, 1), jnp.float32))

        acc = jax.lax.dot_general(p_scr[...], v_ref[0, :, hs],
                                  (((1,), (0,)), ((), ())),
                                  preferred_element_type=jnp.float32)
        o_ref[0, :, hs] = acc / l


def _out_proj_kernel(a_ref, w_ref, o_ref):
    o_ref[0] = jax.lax.dot_general(a_ref[0], w_ref[:], (((1,), (1,)), ((), ())),
                                   preferred_element_type=jnp.float32)


def kernel(x, idx_qk, idx_v, idx_q, idx_k, idx_v2,
           soft_qk, soft_v, soft_q, soft_k, soft_v2,
           feature_qk_neurons, feature_v_neurons, relational_neurons,
           value_neurons, W_O):
    scale = 1.0 / math.sqrt(DH)
    a_qk = feature_qk_neurons[idx_qk] * soft_qk[:, :, None]
    a_v = feature_v_neurons[idx_v] * soft_v[:, :, None]
    r_q = relational_neurons[idx_q] * soft_q[:, :, None]
    r_k = relational_neurons[idx_k] * (soft_k * scale)[:, :, None]
    r_v = value_neurons[idx_v2] * soft_v2[:, :, None]

    n_s = S // BLK_S
    fac_spec = pl.BlockSpec((1, TOPK, D), lambda b, i: (b, 0, 0))
    seq_spec = pl.BlockSpec((1, BLK_S, D), lambda b, i: (b, i, 0))
    q, k, v = pl.pallas_call(
        _qkv_proj_kernel,
        grid=(B, n_s),
        in_specs=[seq_spec, fac_spec, fac_spec, fac_spec, fac_spec, fac_spec],
        out_specs=[seq_spec, seq_spec, seq_spec],
        out_shape=[jax.ShapeDtypeStruct((B, S, D), jnp.bfloat16)] * 3,
    )(x, a_qk, a_v, r_q, r_k, r_v)

    n_q = S // BLK_Q
    attn = pl.pallas_call(
        _flash_kernel,
        grid=(B, n_q),
        in_specs=[
            pl.BlockSpec((1, BLK_Q, D), lambda b, i: (b, i, 0)),
            pl.BlockSpec((1, S, D), lambda b, i: (b, 0, 0)),
            pl.BlockSpec((1, S, D), lambda b, i: (b, 0, 0)),
        ],
        out_specs=pl.BlockSpec((1, BLK_Q, D), lambda b, i: (b, i, 0)),
        out_shape=jax.ShapeDtypeStruct((B, S, D), jnp.float32),
        scratch_shapes=[pltpu.VMEM((BLK_Q, S), jnp.float32),
                        pltpu.VMEM((BLK_Q, S), jnp.bfloat16)],
    )(q, k, v)

    out = pl.pallas_call(
        _out_proj_kernel,
        grid=(B, n_s),
        in_specs=[seq_spec, pl.BlockSpec((D, D), lambda b, i: (0, 0))],
        out_specs=seq_spec,
        out_shape=jax.ShapeDtypeStruct((B, S, D), jnp.float32),
    )(attn, W_O)
    return out


# speedup vs baseline: 1.1485x; 1.1485x over previous
import math

import jax
import jax.numpy as jnp
from jax.experimental import pallas as pl
from jax.experimental.pallas import tpu as pltpu

B = 2
S = 2048
D = 1024
H = 16
DH = 64
POOL = 512
TOPK = 128

BLK_S = 512
BLK_Q = 512
BLK_K = 512


def _qkv_proj_kernel(x_ref, aqk_ref, av_ref, rq_ref, rk_ref, rv_ref,
                     q_ref, k_ref, v_ref):
    x = x_ref[0]
    h_qk = jax.lax.dot_general(x, aqk_ref[0], (((1,), (1,)), ((), ())),
                               preferred_element_type=jnp.float32)
    h_v = jax.lax.dot_general(x, av_ref[0], (((1,), (1,)), ((), ())),
                              preferred_element_type=jnp.float32)
    q_ref[0] = jnp.dot(h_qk, rq_ref[0],
                       preferred_element_type=jnp.float32).astype(jnp.bfloat16)
    k_ref[0] = jnp.dot(h_qk, rk_ref[0],
                       preferred_element_type=jnp.float32).astype(jnp.bfloat16)
    v_ref[0] = jnp.dot(h_v, rv_ref[0],
                       preferred_element_type=jnp.float32).astype(jnp.bfloat16)


def _flash_kernel(q_ref, k_ref, v_ref, o_ref, s_scr):
    i = pl.program_id(1)
    tri = (jax.lax.broadcasted_iota(jnp.int32, (BLK_Q, BLK_K), 1) <=
           jax.lax.broadcasted_iota(jnp.int32, (BLK_Q, BLK_K), 0))

    for h in range(H):
        hs = slice(h * DH, (h + 1) * DH)
        qh = q_ref[0, :, hs]

        def p1(j, m, qh=qh, hs=hs):
            s = jax.lax.dot_general(qh, k_ref[0, pl.ds(j * BLK_K, BLK_K), hs],
                                    (((1,), (1,)), ((), ())),
                                    preferred_element_type=jnp.float32)
            s_scr[:, pl.ds(j * BLK_K, BLK_K)] = s
            return jnp.maximum(m, jnp.max(s, axis=1, keepdims=True))

        m = jax.lax.fori_loop(0, i, p1,
                              jnp.full((BLK_Q, 1), -jnp.inf, jnp.float32))
        s = jax.lax.dot_general(qh, k_ref[0, pl.ds(i * BLK_K, BLK_K), hs],
                                (((1,), (1,)), ((), ())),
                                preferred_element_type=jnp.float32)
        s = jnp.where(tri, s, -1e30)
        s_scr[:, pl.ds(i * BLK_K, BLK_K)] = s
        m = jnp.maximum(m, jnp.max(s, axis=1, keepdims=True))

        def p2(j, carry, m=m, hs=hs):
            acc, l = carry
            p = jnp.exp(s_scr[:, pl.ds(j * BLK_K, BLK_K)] - m)
            l = l + jnp.sum(p, axis=1, keepdims=True)
            acc = acc + jnp.dot(p.astype(jnp.bfloat16),
                                v_ref[0, pl.ds(j * BLK_K, BLK_K), hs],
                                preferred_element_type=jnp.float32)
            return acc, l

        acc, l = jax.lax.fori_loop(0, i + 1, p2,
                                   (jnp.zeros((BLK_Q, DH), jnp.float32),
                                    jnp.zeros((BLK_Q, 1), jnp.float32)))
        o_ref[0, :, hs] = acc / l


def _out_proj_kernel(a_ref, w_ref, o_ref):
    o_ref[0] = jax.lax.dot_general(a_ref[0], w_ref[:], (((1,), (1,)), ((), ())),
                                   preferred_element_type=jnp.float32)


def kernel(x, idx_qk, idx_v, idx_q, idx_k, idx_v2,
           soft_qk, soft_v, soft_q, soft_k, soft_v2,
           feature_qk_neurons, feature_v_neurons, relational_neurons,
           value_neurons, W_O):
    scale = 1.0 / math.sqrt(DH)
    a_qk = feature_qk_neurons[idx_qk] * soft_qk[:, :, None]
    a_v = feature_v_neurons[idx_v] * soft_v[:, :, None]
    r_q = relational_neurons[idx_q] * soft_q[:, :, None]
    r_k = relational_neurons[idx_k] * (soft_k * scale)[:, :, None]
    r_v = value_neurons[idx_v2] * soft_v2[:, :, None]

    n_s = S // BLK_S
    fac_spec = pl.BlockSpec((1, TOPK, D), lambda b, i: (b, 0, 0))
    seq_spec = pl.BlockSpec((1, BLK_S, D), lambda b, i: (b, i, 0))
    q, k, v = pl.pallas_call(
        _qkv_proj_kernel,
        grid=(B, n_s),
        in_specs=[seq_spec, fac_spec, fac_spec, fac_spec, fac_spec, fac_spec],
        out_specs=[seq_spec, seq_spec, seq_spec],
        out_shape=[jax.ShapeDtypeStruct((B, S, D), jnp.bfloat16)] * 3,
    )(x, a_qk, a_v, r_q, r_k, r_v)

    n_q = S // BLK_Q
    attn = pl.pallas_call(
        _flash_kernel,
        grid=(B, n_q),
        in_specs=[
            pl.BlockSpec((1, BLK_Q, D), lambda b, i: (b, i, 0)),
            pl.BlockSpec((1, S, D), lambda b, i: (b, 0, 0)),
            pl.BlockSpec((1, S, D), lambda b, i: (b, 0, 0)),
        ],
        out_specs=pl.BlockSpec((1, BLK_Q, D), lambda b, i: (b, i, 0)),
        out_shape=jax.ShapeDtypeStruct((B, S, D), jnp.float32),
        scratch_shapes=[pltpu.VMEM((BLK_Q, S), jnp.float32)],
    )(q, k, v)

    out = pl.pallas_call(
        _out_proj_kernel,
        grid=(B, n_s),
        in_specs=[seq_spec, pl.BlockSpec((D, D), lambda b, i: (0, 0))],
        out_specs=seq_spec,
        out_shape=jax.ShapeDtypeStruct((B, S, D), jnp.float32),
    )(attn, W_O)
    return out
